# Initial kernel scaffold; baseline (speedup 1.0000x reference)
import jax
import jax.numpy as jnp
from jax.experimental import pallas as pl


def kernel(x, pos, W, b):
    raise NotImplementedError("write your pallas kernel here")


# speedup vs baseline: 20.4066x; 20.4066x over previous
import functools

import jax
import jax.numpy as jnp
from jax.experimental import pallas as pl
from jax.experimental.pallas import tpu as pltpu

_B, _N, _IN_DIM, _OUT_DIM = 2, 2048, 128, 128
_TOP_K = 4
_BR = 256


def _fused_body(pos_blk, pos_t, x_ref, wt_ref, b_ref, out_ref):
    pi = pos_blk[0]
    pj = pos_t[0]

    dx = pi[:, 0:1] - pj[0:1, :]
    dy = pi[:, 1:2] - pj[1:2, :]
    dz = pi[:, 2:3] - pj[2:3, :]
    dsq = dx * dx + dy * dy + dz * dz
    dist = jnp.sqrt(dsq + 1e-8)
    adj = jnp.exp(-(dist * dist) * 0.5)

    col = jax.lax.broadcasted_iota(jnp.int32, adj.shape, 1)
    work = adj
    masked = jnp.zeros_like(adj)
    for _ in range(_TOP_K):
        m = jnp.max(work, axis=1, keepdims=True)
        is_max = work == m
        idx = jnp.min(jnp.where(is_max, col, _N), axis=1, keepdims=True)
        sel = col == idx
        masked = jnp.where(sel, work, masked)
        work = jnp.where(sel, -1.0, work)

    s = jnp.sum(masked, axis=1, keepdims=True)
    adjn = masked / (s + 1e-8)

    agg = jnp.dot(adjn, x_ref[0], preferred_element_type=jnp.float32)
    out = jnp.dot(agg, wt_ref[...], preferred_element_type=jnp.float32)
    out_ref[0] = out + b_ref[...]


@jax.jit
def kernel(x, pos, W, b):
    pos8 = jnp.pad(pos, ((0, 0), (0, 0), (0, 5)))
    pos_t = jnp.transpose(pos8, (0, 2, 1))
    wt = W.T
    b2 = b.reshape(1, _OUT_DIM)

    grid = (_B, _N // _BR)
    out = pl.pallas_call(
        _fused_body,
        grid=grid,
        in_specs=[
            pl.BlockSpec((1, _BR, 8), lambda bi, i: (bi, i, 0)),
            pl.BlockSpec((1, 8, _N), lambda bi, i: (bi, 0, 0)),
            pl.BlockSpec((1, _N, _IN_DIM), lambda bi, i: (bi, 0, 0)),
            pl.BlockSpec((_IN_DIM, _OUT_DIM), lambda bi, i: (0, 0)),
            pl.BlockSpec((1, _OUT_DIM), lambda bi, i: (0, 0)),
        ],
        out_specs=pl.BlockSpec((1, _BR, _OUT_DIM), lambda bi, i: (bi, i, 0)),
        out_shape=jax.ShapeDtypeStruct((_B, _N, _OUT_DIM), jnp.float32),
    )(pos8, pos_t, x, wt, b2)
    return out


# speedup vs baseline: 27.3934x; 1.3424x over previous
import functools

import jax
import jax.numpy as jnp
from jax.experimental import pallas as pl
from jax.experimental.pallas import tpu as pltpu

_B, _N, _IN_DIM, _OUT_DIM = 2, 2048, 128, 128
_TOP_K = 4
_BR = 256


def _aggregate(masked, x_ref, wt_ref, b_ref, out_ref):
    s = jnp.sum(masked, axis=1, keepdims=True)
    agg = jnp.dot(masked, x_ref[0], preferred_element_type=jnp.float32)
    agg = agg / (s + 1e-8)
    out = jnp.dot(agg, wt_ref[...], preferred_element_type=jnp.float32)
    out_ref[0] = out + b_ref[...]


def _fused_body(pos_blk, pos_t, x_ref, wt_ref, b_ref, out_ref):
    pi = pos_blk[0]
    pj = pos_t[0]

    dx = pi[:, 0:1] - pj[0:1, :]
    dy = pi[:, 1:2] - pj[1:2, :]
    dz = pi[:, 2:3] - pj[2:3, :]
    dsq = dx * dx + dy * dy + dz * dz
    dist = jnp.sqrt(dsq + 1e-8)
    adj = jnp.exp(-(dist * dist) * 0.5)

    work = adj
    for _ in range(_TOP_K):
        m = jnp.max(work, axis=1, keepdims=True)
        work = jnp.where(work == m, -1.0, work)
        t_last = m
    sel = adj >= t_last
    cnt = jnp.sum(sel.astype(jnp.int32), axis=1, keepdims=True)
    ok = jnp.all(cnt == _TOP_K)

    @pl.when(ok)
    def _():
        masked = jnp.where(sel, adj, 0.0)
        _aggregate(masked, x_ref, wt_ref, b_ref, out_ref)

    @pl.when(jnp.logical_not(ok))
    def _():
        col = jax.lax.broadcasted_iota(jnp.int32, adj.shape, 1)
        w2 = adj
        masked = jnp.zeros_like(adj)
        for _ in range(_TOP_K):
            m = jnp.max(w2, axis=1, keepdims=True)
            is_max = w2 == m
            idx = jnp.min(jnp.where(is_max, col, _N), axis=1, keepdims=True)
            one = col == idx
            masked = jnp.where(one, w2, masked)
            w2 = jnp.where(one, -1.0, w2)
        _aggregate(masked, x_ref, wt_ref, b_ref, out_ref)


@jax.jit
def kernel(x, pos, W, b):
    pos8 = jnp.pad(pos, ((0, 0), (0, 0), (0, 5)))
    pos_t = jnp.transpose(pos8, (0, 2, 1))
    wt = W.T
    b2 = b.reshape(1, _OUT_DIM)

    grid = (_B, _N // _BR)
    out = pl.pallas_call(
        _fused_body,
        grid=grid,
        in_specs=[
            pl.BlockSpec((1, _BR, 8), lambda bi, i: (bi, i, 0)),
            pl.BlockSpec((1, 8, _N), lambda bi, i: (bi, 0, 0)),
            pl.BlockSpec((1, _N, _IN_DIM), lambda bi, i: (bi, 0, 0)),
            pl.BlockSpec((_IN_DIM, _OUT_DIM), lambda bi, i: (0, 0)),
            pl.BlockSpec((1, _OUT_DIM), lambda bi, i: (0, 0)),
        ],
        out_specs=pl.BlockSpec((1, _BR, _OUT_DIM), lambda bi, i: (bi, i, 0)),
        out_shape=jax.ShapeDtypeStruct((_B, _N, _OUT_DIM), jnp.float32),
    )(pos8, pos_t, x, wt, b2)
    return out


# speedup vs baseline: 28.3677x; 1.0356x over previous
import functools

import jax
import jax.numpy as jnp
from jax.experimental import pallas as pl
from jax.experimental.pallas import tpu as pltpu

_B, _N, _IN_DIM, _OUT_DIM = 2, 2048, 128, 128
_TOP_K = 4
_BR = 512


def _aggregate(masked, x_ref, wt_ref, b_ref, out_ref):
    s = jnp.sum(masked, axis=1, keepdims=True)
    agg = jnp.dot(masked, x_ref[0], preferred_element_type=jnp.float32)
    agg = agg / (s + 1e-8)
    out = jnp.dot(agg, wt_ref[...], preferred_element_type=jnp.float32)
    out_ref[0] = out + b_ref[...]


def _fused_body(pos_blk, pos_t, x_ref, wt_ref, b_ref, out_ref):
    pi = pos_blk[0]
    pj = pos_t[0]

    dx = pi[:, 0:1] - pj[0:1, :]
    dy = pi[:, 1:2] - pj[1:2, :]
    dz = pi[:, 2:3] - pj[2:3, :]
    dsq = dx * dx + dy * dy + dz * dz
    dist = jnp.sqrt(dsq + 1e-8)
    adj = jnp.exp(-(dist * dist) * 0.5)

    work = adj
    for _ in range(_TOP_K):
        m = jnp.max(work, axis=1, keepdims=True)
        work = jnp.where(work == m, -1.0, work)
        t_last = m
    sel = adj >= t_last
    cnt = jnp.sum(sel.astype(jnp.int32), axis=1, keepdims=True)
    ok = jnp.all(cnt == _TOP_K)

    @pl.when(ok)
    def _():
        masked = jnp.where(sel, adj, 0.0)
        _aggregate(masked, x_ref, wt_ref, b_ref, out_ref)

    @pl.when(jnp.logical_not(ok))
    def _():
        col = jax.lax.broadcasted_iota(jnp.int32, adj.shape, 1)
        w2 = adj
        masked = jnp.zeros_like(adj)
        for _ in range(_TOP_K):
            m = jnp.max(w2, axis=1, keepdims=True)
            is_max = w2 == m
            idx = jnp.min(jnp.where(is_max, col, _N), axis=1, keepdims=True)
            one = col == idx
            masked = jnp.where(one, w2, masked)
            w2 = jnp.where(one, -1.0, w2)
        _aggregate(masked, x_ref, wt_ref, b_ref, out_ref)


@jax.jit
def kernel(x, pos, W, b):
    pos8 = jnp.pad(pos, ((0, 0), (0, 0), (0, 5)))
    pos_t = jnp.transpose(pos8, (0, 2, 1))
    wt = W.T
    b2 = b.reshape(1, _OUT_DIM)

    grid = (_B, _N // _BR)
    out = pl.pallas_call(
        _fused_body,
        grid=grid,
        in_specs=[
            pl.BlockSpec((1, _BR, 8), lambda bi, i: (bi, i, 0)),
            pl.BlockSpec((1, 8, _N), lambda bi, i: (bi, 0, 0)),
            pl.BlockSpec((1, _N, _IN_DIM), lambda bi, i: (bi, 0, 0)),
            pl.BlockSpec((_IN_DIM, _OUT_DIM), lambda bi, i: (0, 0)),
            pl.BlockSpec((1, _OUT_DIM), lambda bi, i: (0, 0)),
        ],
        out_specs=pl.BlockSpec((1, _BR, _OUT_DIM), lambda bi, i: (bi, i, 0)),
        out_shape=jax.ShapeDtypeStruct((_B, _N, _OUT_DIM), jnp.float32),
    )(pos8, pos_t, x, wt, b2)
    return out
